# baseline (speedup 1.0000x reference)
import functools

import jax
import jax.numpy as jnp
from jax import lax
from jax.experimental import pallas as pl
from jax.experimental.pallas import tpu as pltpu
from jax.experimental.pallas import tpu_sc as plsc

_HIDDEN = 4096
_NUM_EXPERTS = 64
_TOP_K = 8
_TOKENS = 8192
_BLOCK_T = 512

_NC = 2
_NS = 16
_NW = _NC * _NS
_ROWS_PER_W = _TOKENS // _NW


def _dense_body(x_ref, w_ref, scores_ref):
    logits = lax.dot_general(
        x_ref[...], w_ref[...],
        dimension_numbers=(((1,), (1,)), ((), ())),
        preferred_element_type=jnp.float32,
    )
    m = jnp.max(logits, axis=-1, keepdims=True)
    e = jnp.exp(logits - m)
    scores_ref[...] = e / jnp.sum(e, axis=-1, keepdims=True)


@functools.lru_cache(maxsize=None)
def _make_topk_sc(n_tokens):
    rows_per_w = n_tokens // _NW
    n_in = rows_per_w * _NUM_EXPERTS
    n_out = rows_per_w * _TOP_K

    def _topk_sc_body(scores_hbm, w_hbm, i_hbm, sc_v, w_v, i_v):
        wid = lax.axis_index("s") * _NC + lax.axis_index("c")
        pltpu.sync_copy(scores_hbm.at[pl.ds(wid * n_in, n_in)], sc_v)

        lane = lax.iota(jnp.int32, 16)
        lo = lane < _TOP_K

        def merge(a, b):
            ck = jnp.where(lo, a[0], lax.rev(b[0], (0,)))
            cv = jnp.where(lo, a[1], lax.rev(b[1], (0,)))
            return plsc.sort_key_val(ck, cv, descending=True)

        def row_body(r, carry):
            base = r * _NUM_EXPERTS
            svs = [
                plsc.sort_key_val(
                    sc_v[pl.ds(base + 16 * j, 16)], lane + 16 * j, descending=True
                )
                for j in range(4)
            ]
            fk, fv = merge(merge(svs[0], svs[1]), merge(svs[2], svs[3]))
            w_v[pl.ds(r * _TOP_K, 16)] = fk
            i_v[pl.ds(r * _TOP_K, 16)] = fv
            return carry

        lax.fori_loop(0, rows_per_w, row_body, 0)
        pltpu.sync_copy(w_v.at[pl.ds(0, n_out)], w_hbm.at[pl.ds(wid * n_out, n_out)])
        pltpu.sync_copy(i_v.at[pl.ds(0, n_out)], i_hbm.at[pl.ds(wid * n_out, n_out)])

    return pl.kernel(
        _topk_sc_body,
        mesh=plsc.VectorSubcoreMesh(core_axis_name="c", subcore_axis_name="s"),
        compiler_params=pltpu.CompilerParams(needs_layout_passes=False),
        out_type=[
            jax.ShapeDtypeStruct((n_tokens * _TOP_K,), jnp.float32),
            jax.ShapeDtypeStruct((n_tokens * _TOP_K,), jnp.int32),
        ],
        scratch_types=[
            pltpu.VMEM((n_in,), jnp.float32),
            pltpu.VMEM((n_out + 8,), jnp.float32),
            pltpu.VMEM((n_out + 8,), jnp.int32),
        ],
    )


_NUM_CHUNKS = 4


def _dense_chunk(x_chunk, W):
    tokens = x_chunk.shape[0]
    grid = tokens // _BLOCK_T
    return pl.pallas_call(
        _dense_body,
        grid=(grid,),
        in_specs=[
            pl.BlockSpec((_BLOCK_T, _HIDDEN), lambda i: (i, 0)),
            pl.BlockSpec((_NUM_EXPERTS, _HIDDEN), lambda i: (0, 0)),
        ],
        out_specs=pl.BlockSpec((_BLOCK_T, _NUM_EXPERTS), lambda i: (i, 0)),
        out_shape=jax.ShapeDtypeStruct((tokens, _NUM_EXPERTS), jnp.float32),
    )(x_chunk, W)


@jax.jit
def kernel(x, W):
    tokens = x.shape[0]
    chunk = tokens // _NUM_CHUNKS
    score_chunks = []
    w_chunks = []
    i_chunks = []
    for c in range(_NUM_CHUNKS):
        s = _dense_chunk(lax.slice(x, (c * chunk, 0), ((c + 1) * chunk, _HIDDEN)), W)
        wf, it = _make_topk_sc(chunk)(s.reshape(-1))
        score_chunks.append(s)
        w_chunks.append(wf.reshape(chunk, _TOP_K))
        i_chunks.append(it.reshape(chunk, _TOP_K))
    return (
        jnp.concatenate(score_chunks, axis=0),
        jnp.concatenate(w_chunks, axis=0),
        jnp.concatenate(i_chunks, axis=0),
    )


# speedup vs baseline: 1.8919x; 1.8919x over previous
import functools

import jax
import jax.numpy as jnp
from jax import lax
from jax.experimental import pallas as pl
from jax.experimental.pallas import tpu as pltpu
from jax.experimental.pallas import tpu_sc as plsc

_HIDDEN = 4096
_NUM_EXPERTS = 64
_TOP_K = 8
_TOKENS = 8192
_BLOCK_T = 512

_NC = 2
_NS = 16
_NW = _NC * _NS
_ROWS_PER_W = _TOKENS // _NW


def _dense_body(x_ref, w_ref, scores_ref):
    logits = lax.dot_general(
        x_ref[...], w_ref[...],
        dimension_numbers=(((1,), (1,)), ((), ())),
        preferred_element_type=jnp.float32,
    )
    m = jnp.max(logits, axis=-1, keepdims=True)
    e = jnp.exp(logits - m)
    scores_ref[...] = e / jnp.sum(e, axis=-1, keepdims=True)


@functools.lru_cache(maxsize=None)
def _make_topk_sc(n_tokens):
    rows_per_w = n_tokens // _NW
    n_in = rows_per_w * _NUM_EXPERTS
    n_out = rows_per_w * _TOP_K

    def _topk_sc_body(scores_hbm, w_hbm, i_hbm, sc_v, w_v, i_v):
        wid = lax.axis_index("s") * _NC + lax.axis_index("c")
        pltpu.sync_copy(scores_hbm.at[pl.ds(wid * n_in, n_in)], sc_v)

        lane = lax.iota(jnp.int32, 16)
        lo = lane < _TOP_K

        def merge(a, b):
            ck = jnp.where(lo, a[0], lax.rev(b[0], (0,)))
            cv = jnp.where(lo, a[1], lax.rev(b[1], (0,)))
            return plsc.sort_key_val(ck, cv, descending=True)

        def row_body(r, carry):
            base = r * _NUM_EXPERTS
            svs = [
                plsc.sort_key_val(
                    sc_v[pl.ds(base + 16 * j, 16)], lane + 16 * j, descending=True
                )
                for j in range(4)
            ]
            fk, fv = merge(merge(svs[0], svs[1]), merge(svs[2], svs[3]))
            w_v[pl.ds(r * _TOP_K, 16)] = fk
            i_v[pl.ds(r * _TOP_K, 16)] = fv
            return carry

        lax.fori_loop(0, rows_per_w, row_body, 0)
        pltpu.sync_copy(w_v.at[pl.ds(0, n_out)], w_hbm.at[pl.ds(wid * n_out, n_out)])
        pltpu.sync_copy(i_v.at[pl.ds(0, n_out)], i_hbm.at[pl.ds(wid * n_out, n_out)])

    return pl.kernel(
        _topk_sc_body,
        mesh=plsc.VectorSubcoreMesh(core_axis_name="c", subcore_axis_name="s"),
        compiler_params=pltpu.CompilerParams(needs_layout_passes=False),
        out_type=[
            jax.ShapeDtypeStruct((n_tokens * _TOP_K,), jnp.float32),
            jax.ShapeDtypeStruct((n_tokens * _TOP_K,), jnp.int32),
        ],
        scratch_types=[
            pltpu.VMEM((n_in,), jnp.float32),
            pltpu.VMEM((n_out + 8,), jnp.float32),
            pltpu.VMEM((n_out + 8,), jnp.int32),
        ],
    )


_NUM_CHUNKS = 4


def _dense_chunk(x, W, c, chunk):
    grid = chunk // _BLOCK_T
    blocks_per_chunk = grid

    def x_map(i, c=c):
        return (c * blocks_per_chunk + i, 0)

    return pl.pallas_call(
        _dense_body,
        grid=(grid,),
        in_specs=[
            pl.BlockSpec((_BLOCK_T, _HIDDEN), x_map),
            pl.BlockSpec((_NUM_EXPERTS, _HIDDEN), lambda i: (0, 0)),
        ],
        out_specs=pl.BlockSpec((_BLOCK_T, _NUM_EXPERTS), lambda i: (i, 0)),
        out_shape=jax.ShapeDtypeStruct((chunk, _NUM_EXPERTS), jnp.float32),
    )(x, W)


@jax.jit
def kernel(x, W):
    tokens = x.shape[0]
    chunk = tokens // _NUM_CHUNKS
    score_chunks = []
    w_chunks = []
    i_chunks = []
    for c in range(_NUM_CHUNKS):
        s = _dense_chunk(x, W, c, chunk)
        wf, it = _make_topk_sc(chunk)(s.reshape(-1))
        score_chunks.append(s)
        w_chunks.append(wf.reshape(chunk, _TOP_K))
        i_chunks.append(it.reshape(chunk, _TOP_K))
    return (
        jnp.concatenate(score_chunks, axis=0),
        jnp.concatenate(w_chunks, axis=0),
        jnp.concatenate(i_chunks, axis=0),
    )


# speedup vs baseline: 2.2823x; 1.2064x over previous
import functools

import jax
import jax.numpy as jnp
from jax import lax
from jax.experimental import pallas as pl
from jax.experimental.pallas import tpu as pltpu
from jax.experimental.pallas import tpu_sc as plsc

_HIDDEN = 4096
_NUM_EXPERTS = 64
_TOP_K = 8
_TOKENS = 8192
_BLOCK_T = 512

_NC = 2
_NS = 16
_NW = _NC * _NS
_ROWS_PER_W = _TOKENS // _NW


def _dense_body(x_ref, w_ref, scores_ref):
    logits = lax.dot_general(
        x_ref[...], w_ref[...],
        dimension_numbers=(((1,), (1,)), ((), ())),
        preferred_element_type=jnp.float32,
    )
    m = jnp.max(logits, axis=-1, keepdims=True)
    e = jnp.exp(logits - m)
    scores_ref[...] = e / jnp.sum(e, axis=-1, keepdims=True)


@functools.lru_cache(maxsize=None)
def _make_topk_sc(n_tokens):
    rows_per_w = n_tokens // _NW
    n_in = rows_per_w * _NUM_EXPERTS
    n_out = rows_per_w * _TOP_K

    def _topk_sc_body(scores_hbm, w_hbm, i_hbm, sc_v, w_v, i_v):
        wid = lax.axis_index("s") * _NC + lax.axis_index("c")
        pltpu.sync_copy(scores_hbm.at[pl.ds(wid * n_in, n_in)], sc_v)

        lane = lax.iota(jnp.int32, 16)
        lo = lane < _TOP_K

        def merge(a, b):
            ck = jnp.where(lo, a[0], lax.rev(b[0], (0,)))
            cv = jnp.where(lo, a[1], lax.rev(b[1], (0,)))
            return plsc.sort_key_val(ck, cv, descending=True)

        @plsc.parallel_loop(0, rows_per_w, 1, unroll=8)
        def row_body(r):
            base = r * _NUM_EXPERTS
            svs = [
                plsc.sort_key_val(
                    sc_v[pl.ds(base + 16 * j, 16)], lane + 16 * j, descending=True
                )
                for j in range(4)
            ]
            fk, fv = merge(merge(svs[0], svs[1]), merge(svs[2], svs[3]))
            plsc.store_compressed(w_v.at[pl.ds(r * _TOP_K, 16)], fk, mask=lo)
            plsc.store_compressed(i_v.at[pl.ds(r * _TOP_K, 16)], fv, mask=lo)
        pltpu.sync_copy(w_v.at[pl.ds(0, n_out)], w_hbm.at[pl.ds(wid * n_out, n_out)])
        pltpu.sync_copy(i_v.at[pl.ds(0, n_out)], i_hbm.at[pl.ds(wid * n_out, n_out)])

    return pl.kernel(
        _topk_sc_body,
        mesh=plsc.VectorSubcoreMesh(core_axis_name="c", subcore_axis_name="s"),
        compiler_params=pltpu.CompilerParams(needs_layout_passes=False),
        out_type=[
            jax.ShapeDtypeStruct((n_tokens * _TOP_K,), jnp.float32),
            jax.ShapeDtypeStruct((n_tokens * _TOP_K,), jnp.int32),
        ],
        scratch_types=[
            pltpu.VMEM((n_in,), jnp.float32),
            pltpu.VMEM((n_out + 8,), jnp.float32),
            pltpu.VMEM((n_out + 8,), jnp.int32),
        ],
    )


_NUM_CHUNKS = 1


def _dense_chunk(x, W, c, chunk):
    grid = chunk // _BLOCK_T
    blocks_per_chunk = grid

    def x_map(i, c=c):
        return (c * blocks_per_chunk + i, 0)

    return pl.pallas_call(
        _dense_body,
        grid=(grid,),
        in_specs=[
            pl.BlockSpec((_BLOCK_T, _HIDDEN), x_map),
            pl.BlockSpec((_NUM_EXPERTS, _HIDDEN), lambda i: (0, 0)),
        ],
        out_specs=pl.BlockSpec((_BLOCK_T, _NUM_EXPERTS), lambda i: (i, 0)),
        out_shape=jax.ShapeDtypeStruct((chunk, _NUM_EXPERTS), jnp.float32),
    )(x, W)


@jax.jit
def kernel(x, W):
    tokens = x.shape[0]
    if _NUM_CHUNKS == 1:
        s = _dense_chunk(x, W, 0, tokens)
        wf, it = _make_topk_sc(tokens)(s.reshape(-1))
        return (s, wf.reshape(tokens, _TOP_K), it.reshape(tokens, _TOP_K))
    chunk = tokens // _NUM_CHUNKS
    score_chunks = []
    w_chunks = []
    i_chunks = []
    for c in range(_NUM_CHUNKS):
        s = _dense_chunk(x, W, c, chunk)
        wf, it = _make_topk_sc(chunk)(s.reshape(-1))
        score_chunks.append(s)
        w_chunks.append(wf.reshape(chunk, _TOP_K))
        i_chunks.append(it.reshape(chunk, _TOP_K))
    return (
        jnp.concatenate(score_chunks, axis=0),
        jnp.concatenate(w_chunks, axis=0),
        jnp.concatenate(i_chunks, axis=0),
    )
